# baseline (speedup 1.0000x reference)
import functools

import jax
import jax.numpy as jnp
from jax import lax
from jax.experimental import pallas as pl
from jax.experimental.pallas import tpu as pltpu
from jax.experimental.pallas import tpu_sc as plsc

N_ROWS = 1_000_000
ROW_W = 97
PROW = 128
NBITS = 48
CHUNK = 400
NCHUNKS = N_ROWS // CHUNK
NWORKERS = 32
GROUPS = CHUNK // 8
SLOT_STRIDE = CHUNK * PROW
_BASE_CHUNKS = NCHUNKS // NWORKERS
_EXTRA = NCHUNKS - _BASE_CHUNKS * NWORKERS


def _sc_body(preds_hbm, tgt_hbm, out_hbm, pbuf, tbuf, accbuf, psem, tsem):
    wid = (lax.axis_index("s") * 2 + lax.axis_index("c")).astype(jnp.int32)
    nc = jnp.int32(_BASE_CHUNKS) + jnp.where(wid < _EXTRA, jnp.int32(1),
                                             jnp.int32(0))
    lane = lax.iota(jnp.int32, 16)
    zidx = jnp.zeros((16,), jnp.int32)
    low = lane < 8
    rvec7 = jnp.where(low, lane, 15 - lane)
    off24 = jnp.where(low, jnp.int32(0), jnp.int32(24))
    lm1 = jnp.where(low, jnp.int32(1), jnp.int32(0))

    def start_dma(i, slot):
        c = wid + NWORKERS * i
        pltpu.make_async_copy(
            preds_hbm.at[pl.ds(c * CHUNK, CHUNK), :], pbuf.at[slot], psem
        ).start()
        pltpu.make_async_copy(
            tgt_hbm.at[pl.ds(c * CHUNK, CHUNK)],
            tbuf.at[pl.ds(slot * CHUNK, CHUNK)], tsem).start()

    def wait_dma(slot):
        pltpu.make_async_copy(
            preds_hbm.at[pl.ds(0, CHUNK), :], pbuf.at[slot], psem).wait()
        pltpu.make_async_copy(
            tgt_hbm.at[pl.ds(0, CHUNK)],
            tbuf.at[pl.ds(slot * CHUNK, CHUNK)], tsem).wait()

    start_dma(jnp.int32(0), jnp.int32(0))

    def chunk_body(i, acc):
        p = jnp.bitwise_and(i, jnp.int32(1))
        wait_dma(p)

        @pl.when(i + 1 < nc)
        def _():
            start_dma(i + 1, jnp.int32(1) - p)

        pbase = p * SLOT_STRIDE
        tbase = p * CHUNK

        def group_body(g, acc):
            rbase = pbase + (g * 8 + rvec7) * PROW
            sgn_f = plsc.load_gather(pbuf, [zidx, zidx, rbase + (ROW_W - 1)])
            sgn = sgn_f >= 0.0
            cbs = rbase + jnp.where(sgn, jnp.int32(0), jnp.int32(NBITS))
            cb = cbs + off24
            maxi = jnp.full((16,), jnp.int32(0x7FFFFFFF))
            m0 = maxi
            m1 = maxi
            i0 = jnp.zeros((16,), jnp.int32)
            i1 = jnp.zeros((16,), jnp.int32)
            accb = jnp.zeros((16,), jnp.int32)
            for k in range(24):
                v = plsc.load_gather(pbuf, [zidx, zidx, cb + k])
                vb = plsc.bitcast(v, jnp.int32)
                ci = vb & jnp.int32(0x7FFFFFFF)
                sneg = vb >> 31
                accb = accb + (sneg & jnp.int32(1 << (23 - k)))
                kcv = off24 + jnp.int32(k)
                d1 = ci - m1
                lt = d1 >> 31
                m1 = m1 + (lt & d1)
                i1 = i1 + (lt & (kcv - i1))
                d0 = m1 - m0
                sw = d0 >> 31
                dm = sw & d0
                m0 = m0 + dm
                m1 = m1 - dm
                di = sw & (i1 - i0)
                i0 = i0 + di
                i1 = i1 - di

            pm0 = jnp.flip(m0)
            pi0 = jnp.flip(i0)
            pm1 = jnp.flip(m1)
            pi1 = jnp.flip(i1)

            def lex_lt(xv, xi, yv, yi):
                return (xv < yv) | ((xv == yv) & (xi < yi))

            w = lex_lt(pm0, pi0, m0, i0)
            mm0 = jnp.where(w, pm0, m0)
            ii0 = jnp.where(w, pi0, i0)
            cxv = jnp.where(w, pm1, m1)
            cxi = jnp.where(w, pi1, i1)
            cyv = jnp.where(w, m0, pm0)
            cyi = jnp.where(w, i0, pi0)
            w2 = lex_lt(cyv, cyi, cxv, cxi)
            i0 = ii0
            i1 = jnp.where(w2, cyi, cxi)
            del mm0

            paccb = jnp.flip(accb)
            hi = jnp.int32(0xFFFFFF) - jnp.where(low, accb, paccb)
            lo = jnp.int32(0xFFFFFF) - jnp.where(low, paccb, accb)

            def wsplit(idx):
                sh_h = jnp.maximum(23 - idx, 0)
                wh = jnp.where(idx < 24, jnp.int32(1) << sh_h, 0)
                sh_l = jnp.where(idx >= 24, 47 - idx, 0)
                wl = jnp.where(idx >= 24, jnp.int32(1) << sh_l, 0)
                return wh, wl

            wh0, wl0 = wsplit(i0)
            wh1, wl1 = wsplit(i1)
            v0 = plsc.load_gather(pbuf, [zidx, zidx, cbs + i0])
            v1 = plsc.load_gather(pbuf, [zidx, zidx, cbs + i1])
            nm0 = jnp.bitwise_not(plsc.bitcast(v0, jnp.int32)) >> 31
            nm1 = jnp.bitwise_not(plsc.bitcast(v1, jnp.int32)) >> 31
            hic = hi - (nm0 & wh0) - (nm1 & wh1)
            loc = lo - (nm0 & wl0) - (nm1 & wl1)

            tv = plsc.load_gather(tbuf, [tbase + g * 8 + rvec7])
            th = tv >> 24
            tl = tv & 0xFFFFFF
            alo_raw = tl - 128
            ahi_p = th + (alo_raw >> 24)
            alo_p = alo_raw & 0xFFFFFF
            blo_raw = tl + 128
            bhi_p = th + (blo_raw >> 24)
            blo_p = blo_raw & 0xFFFFFF
            neg_ok = (th == 0) & (tl <= 128)
            bhi_n = jnp.where(neg_ok, jnp.int32(0), jnp.int32(-1))
            blo_n = 128 - tl
            ahi = jnp.where(sgn, ahi_p, jnp.int32(-1))
            alo = jnp.where(sgn, alo_p, jnp.int32(0))
            bhi = jnp.where(sgn, bhi_p, bhi_n)
            blo = jnp.where(sgn, blo_p, blo_n)

            def inrange(h, l):
                ge = (h > ahi) | ((h == ahi) & (l >= alo))
                le = (h < bhi) | ((h == bhi) & (l <= blo))
                return ge & le

            ok = inrange(hic, loc)
            ok = ok | inrange(hic + wh1, loc + wl1)
            ok = ok | inrange(hic + wh0, loc + wl0)
            ok = ok | inrange(hic + wh0 + wh1, loc + wl0 + wl1)
            return acc + jnp.where(ok, lm1, 0)

        return lax.fori_loop(jnp.int32(0), jnp.int32(GROUPS), group_body, acc)

    acc = lax.fori_loop(jnp.int32(0), nc, chunk_body,
                        jnp.zeros((16,), jnp.int32))
    accbuf[...] = acc
    pltpu.sync_copy(accbuf, out_hbm.at[pl.ds(wid * 16, 16)])


@jax.jit
def _sc_accuracy(preds, tgt32):
    mesh = plsc.VectorSubcoreMesh(core_axis_name="c", subcore_axis_name="s")
    run = functools.partial(
        pl.kernel,
        mesh=mesh,
        out_type=jax.ShapeDtypeStruct((NWORKERS * 16,), jnp.int32),
        scratch_types=[
            pltpu.VMEM((2, CHUNK, ROW_W), jnp.float32),
            pltpu.VMEM((2 * CHUNK,), jnp.int32),
            pltpu.VMEM((16,), jnp.int32),
            pltpu.SemaphoreType.DMA,
            pltpu.SemaphoreType.DMA,
        ],
        compiler_params=pltpu.CompilerParams(
            needs_layout_passes=False, use_tc_tiling_on_sc=True),
    )(_sc_body)
    return run(preds, tgt32)


def kernel(preds, target):
    tgt32 = target.astype(jnp.int32)
    partials = _sc_accuracy(preds, tgt32)
    total = jnp.sum(partials, dtype=jnp.int32)
    return total.astype(jnp.float32) / jnp.float32(N_ROWS)


# speedup vs baseline: 1.0047x; 1.0047x over previous
import functools

import jax
import jax.numpy as jnp
from jax import lax
from jax.experimental import pallas as pl
from jax.experimental.pallas import tpu as pltpu
from jax.experimental.pallas import tpu_sc as plsc

N_ROWS = 1_000_000
ROW_W = 97
PROW = 128
NBITS = 48
CHUNK = 320
NCHUNKS = N_ROWS // CHUNK
NWORKERS = 32
GROUPS = CHUNK // 8
SLOT_STRIDE = CHUNK * PROW
_BASE_CHUNKS = NCHUNKS // NWORKERS
_EXTRA = NCHUNKS - _BASE_CHUNKS * NWORKERS


def _sc_body(preds_hbm, tgt_hbm, out_hbm, pbuf, tbuf, accbuf, psem, tsem):
    wid = (lax.axis_index("s") * 2 + lax.axis_index("c")).astype(jnp.int32)
    nc = jnp.int32(_BASE_CHUNKS) + jnp.where(wid < _EXTRA, jnp.int32(1),
                                             jnp.int32(0))
    lane = lax.iota(jnp.int32, 16)
    zidx = jnp.zeros((16,), jnp.int32)
    low = lane < 8
    rvec7 = jnp.where(low, lane, 15 - lane)
    off24 = jnp.where(low, jnp.int32(0), jnp.int32(24))
    lm1 = jnp.where(low, jnp.int32(1), jnp.int32(0))

    def start_dma(i, slot):
        c = wid + NWORKERS * i
        pltpu.make_async_copy(
            preds_hbm.at[pl.ds(c * CHUNK, CHUNK), :], pbuf.at[slot], psem
        ).start()
        pltpu.make_async_copy(
            tgt_hbm.at[pl.ds(c * CHUNK, CHUNK)],
            tbuf.at[pl.ds(slot * CHUNK, CHUNK)], tsem).start()

    def wait_dma(slot):
        pltpu.make_async_copy(
            preds_hbm.at[pl.ds(0, CHUNK), :], pbuf.at[slot], psem).wait()
        pltpu.make_async_copy(
            tgt_hbm.at[pl.ds(0, CHUNK)],
            tbuf.at[pl.ds(slot * CHUNK, CHUNK)], tsem).wait()

    start_dma(jnp.int32(0), jnp.int32(0))

    def chunk_body(i, acc):
        p = jnp.bitwise_and(i, jnp.int32(1))
        wait_dma(p)

        @pl.when(i + 1 < nc)
        def _():
            start_dma(i + 1, jnp.int32(1) - p)

        pbase = p * SLOT_STRIDE
        tbase = p * CHUNK

        def group_body(g, acc):
            rbase = pbase + (g * 8 + rvec7) * PROW
            sgn_f = plsc.load_gather(pbuf, [zidx, zidx, rbase + (ROW_W - 1)])
            sgn = sgn_f >= 0.0
            cbs = rbase + jnp.where(sgn, jnp.int32(0), jnp.int32(NBITS))
            cb = cbs + off24
            maxi = jnp.full((16,), jnp.int32(0x7FFFFFFF))
            m0 = maxi
            m1 = maxi
            i0 = jnp.zeros((16,), jnp.int32)
            i1 = jnp.zeros((16,), jnp.int32)
            accb = jnp.zeros((16,), jnp.int32)
            for k in range(24):
                v = plsc.load_gather(pbuf, [zidx, zidx, cb + k])
                vb = plsc.bitcast(v, jnp.int32)
                ci = vb & jnp.int32(0x7FFFFFFF)
                sneg = vb >> 31
                accb = accb + (sneg & jnp.int32(1 << (23 - k)))
                kcv = off24 + jnp.int32(k)
                d1 = ci - m1
                lt = d1 >> 31
                m1 = m1 + (lt & d1)
                i1 = i1 + (lt & (kcv - i1))
                d0 = m1 - m0
                sw = d0 >> 31
                dm = sw & d0
                m0 = m0 + dm
                m1 = m1 - dm
                di = sw & (i1 - i0)
                i0 = i0 + di
                i1 = i1 - di

            pm0 = jnp.flip(m0)
            pi0 = jnp.flip(i0)
            pm1 = jnp.flip(m1)
            pi1 = jnp.flip(i1)

            def lex_lt(xv, xi, yv, yi):
                return (xv < yv) | ((xv == yv) & (xi < yi))

            w = lex_lt(pm0, pi0, m0, i0)
            mm0 = jnp.where(w, pm0, m0)
            ii0 = jnp.where(w, pi0, i0)
            cxv = jnp.where(w, pm1, m1)
            cxi = jnp.where(w, pi1, i1)
            cyv = jnp.where(w, m0, pm0)
            cyi = jnp.where(w, i0, pi0)
            w2 = lex_lt(cyv, cyi, cxv, cxi)
            i0 = ii0
            i1 = jnp.where(w2, cyi, cxi)
            del mm0

            paccb = jnp.flip(accb)
            hi = jnp.int32(0xFFFFFF) - jnp.where(low, accb, paccb)
            lo = jnp.int32(0xFFFFFF) - jnp.where(low, paccb, accb)

            def wsplit(idx):
                sh_h = jnp.maximum(23 - idx, 0)
                wh = jnp.where(idx < 24, jnp.int32(1) << sh_h, 0)
                sh_l = jnp.where(idx >= 24, 47 - idx, 0)
                wl = jnp.where(idx >= 24, jnp.int32(1) << sh_l, 0)
                return wh, wl

            wh0, wl0 = wsplit(i0)
            wh1, wl1 = wsplit(i1)
            v0 = plsc.load_gather(pbuf, [zidx, zidx, cbs + i0])
            v1 = plsc.load_gather(pbuf, [zidx, zidx, cbs + i1])
            nm0 = jnp.bitwise_not(plsc.bitcast(v0, jnp.int32)) >> 31
            nm1 = jnp.bitwise_not(plsc.bitcast(v1, jnp.int32)) >> 31
            hic = hi - (nm0 & wh0) - (nm1 & wh1)
            loc = lo - (nm0 & wl0) - (nm1 & wl1)

            tv = plsc.load_gather(tbuf, [tbase + g * 8 + rvec7])
            th = tv >> 24
            tl = tv & 0xFFFFFF
            alo_raw = tl - 128
            ahi_p = th + (alo_raw >> 24)
            alo_p = alo_raw & 0xFFFFFF
            blo_raw = tl + 128
            bhi_p = th + (blo_raw >> 24)
            blo_p = blo_raw & 0xFFFFFF
            neg_ok = (th == 0) & (tl <= 128)
            bhi_n = jnp.where(neg_ok, jnp.int32(0), jnp.int32(-1))
            blo_n = 128 - tl
            ahi = jnp.where(sgn, ahi_p, jnp.int32(-1))
            alo = jnp.where(sgn, alo_p, jnp.int32(0))
            bhi = jnp.where(sgn, bhi_p, bhi_n)
            blo = jnp.where(sgn, blo_p, blo_n)

            def inrange(h, l):
                ge = (h > ahi) | ((h == ahi) & (l >= alo))
                le = (h < bhi) | ((h == bhi) & (l <= blo))
                return ge & le

            ok = inrange(hic, loc)
            ok = ok | inrange(hic + wh1, loc + wl1)
            ok = ok | inrange(hic + wh0, loc + wl0)
            ok = ok | inrange(hic + wh0 + wh1, loc + wl0 + wl1)
            return acc + jnp.where(ok, lm1, 0)

        return lax.fori_loop(jnp.int32(0), jnp.int32(GROUPS), group_body, acc)

    acc = lax.fori_loop(jnp.int32(0), nc, chunk_body,
                        jnp.zeros((16,), jnp.int32))
    accbuf[...] = acc
    pltpu.sync_copy(accbuf, out_hbm.at[pl.ds(wid * 16, 16)])


@jax.jit
def _sc_accuracy(preds, tgt32):
    mesh = plsc.VectorSubcoreMesh(core_axis_name="c", subcore_axis_name="s")
    run = functools.partial(
        pl.kernel,
        mesh=mesh,
        out_type=jax.ShapeDtypeStruct((NWORKERS * 16,), jnp.int32),
        scratch_types=[
            pltpu.VMEM((2, CHUNK, ROW_W), jnp.float32),
            pltpu.VMEM((2 * CHUNK,), jnp.int32),
            pltpu.VMEM((16,), jnp.int32),
            pltpu.SemaphoreType.DMA,
            pltpu.SemaphoreType.DMA,
        ],
        compiler_params=pltpu.CompilerParams(
            needs_layout_passes=False, use_tc_tiling_on_sc=True),
    )(_sc_body)
    return run(preds, tgt32)


def kernel(preds, target):
    tgt32 = target.astype(jnp.int32)
    partials = _sc_accuracy(preds, tgt32)
    total = jnp.sum(partials, dtype=jnp.int32)
    return total.astype(jnp.float32) / jnp.float32(N_ROWS)
